# Initial kernel scaffold; baseline (speedup 1.0000x reference)
import functools

import jax
import jax.numpy as jnp
import numpy as np
from jax import lax
from jax.experimental import pallas as pl
from jax.experimental.pallas import tpu as pltpu
from jax.experimental.pallas import tpu_sc as plsc

GRID_N = 200
ROW_PAD = 256
NORM = float(25000 * 80)
CLIP_MAX = np.float32(GRID_N - 1 - 1e-6)

NC = 2
NS = 16
NW = NC * NS
L = 16

T = 8192
GROUPS = T // L


def _sc_hist_body(points_hbm, out_hbm, bufa, bufb, hist, sema, semb):
  n = points_hbm.shape[0]
  num_tiles = -(-n // T)
  tiles_per_worker = -(-num_tiles // NW)
  last_base = n - T

  wid = lax.axis_index("c") * NS + lax.axis_index("s")

  iota = lax.iota(jnp.int32, L)
  col_x = jnp.zeros((L,), jnp.int32)
  col_y = jnp.ones((L,), jnp.int32)

  def _zero(i, c):
    hist[i >> 4, pl.ds((i & 15) * L, L)] = jnp.zeros((L,), jnp.float32)
    return c
  lax.fori_loop(0, (GRID_N * ROW_PAD) // L, _zero, 0)

  bufs = (bufa, bufb)
  sems = (sema, semb)

  def _base_for(t):
    tile = wid + t * NW
    return jnp.minimum(tile * T, last_base)

  handles = [None, None]
  handles[0] = pltpu.async_copy(
      points_hbm.at[pl.ds(_base_for(0), T)], bufs[0], sems[0])

  for t in range(tiles_per_worker):
    b = t % 2
    if t + 1 < tiles_per_worker:
      nb = (t + 1) % 2
      handles[nb] = pltpu.async_copy(
          points_hbm.at[pl.ds(_base_for(t + 1), T)], bufs[nb], sems[nb])
    handles[b].wait()

    tile_start = (wid + t * NW) * T
    base = jnp.minimum(tile_start, last_base)
    off = tile_start - base
    buf = bufs[b]

    def _group(g, c, buf=buf, off=off):
      local = g * L + iota
      xv = plsc.load_gather(buf, [local, col_x])
      yv = plsc.load_gather(buf, [local, col_y])
      xv = jnp.clip(xv, 0.0, CLIP_MAX)
      yv = jnp.clip(yv, 0.0, CLIP_MAX)
      xi = (xv + 0.5).astype(jnp.int32)
      yi = (yv + 0.5).astype(jnp.int32)
      flat = xi * ROW_PAD + yi
      valid = local >= off
      counts, last_occ = plsc.scan_count(flat, mask=valid)
      plsc.addupdate_scatter(
          hist, [xi, yi], counts.astype(jnp.float32), mask=last_occ)
      return c
    lax.fori_loop(0, GROUPS, _group, 0)

  pltpu.sync_copy(hist, out_hbm.at[wid])


def _merge_body(partials_ref, grid_ref, out_ref):
  s = jnp.sum(partials_ref[...], axis=0)
  prob = s[:, :GRID_N] / NORM
  out_ref[...] = jnp.where(grid_ref[...] != 0, 0.0, prob)


@jax.jit
def kernel(all_points, grid):
  sc_hist = pl.kernel(
      _sc_hist_body,
      out_type=jax.ShapeDtypeStruct((NW, GRID_N, ROW_PAD), jnp.float32),
      mesh=plsc.VectorSubcoreMesh(
          core_axis_name="c", subcore_axis_name="s",
          num_cores=NC, num_subcores=NS),
      scratch_types=[
          pltpu.VMEM((T, 2), jnp.float32),
          pltpu.VMEM((T, 2), jnp.float32),
          pltpu.VMEM((GRID_N, ROW_PAD), jnp.float32),
          pltpu.SemaphoreType.DMA,
          pltpu.SemaphoreType.DMA,
      ],
  )
  partials = sc_hist(all_points)
  merge = pl.pallas_call(
      _merge_body,
      out_shape=jax.ShapeDtypeStruct((GRID_N, GRID_N), jnp.float32),
  )
  return merge(partials, grid)


# speedup vs baseline: 3.8415x; 3.8415x over previous
import jax
import jax.numpy as jnp
import numpy as np
from jax import lax
from jax.experimental import pallas as pl
from jax.experimental.pallas import tpu as pltpu
from jax.experimental.pallas import tpu_sc as plsc

GRID_N = 200
NBINS = GRID_N * GRID_N
NORM = float(25000 * 80)
CLIP_MAX = np.float32(GRID_N - 1 - 1e-6)

NC = 2
NS = 16
L = 16

T = 8192


def _sc_hist_body(points_hbm, out_hbm, bufa, bufb, hist, sema, semb):
  nw = NC * NS
  groups = T // L
  n = points_hbm.shape[0] // 2
  num_tiles = -(-n // T)
  tiles_per_worker = -(-num_tiles // nw)
  last_base = n - T

  wid = lax.axis_index("c") * NS + lax.axis_index("s")

  iota = lax.iota(jnp.int32, L)

  def _zero(i, c):
    hist[pl.ds(i * L, L)] = jnp.zeros((L,), jnp.float32)
    return c
  lax.fori_loop(0, NBINS // L, _zero, 0)

  bufs = (bufa, bufb)
  sems = (sema, semb)

  def _start(t):
    base = jnp.minimum((wid + t * nw) * T, last_base)
    return pltpu.async_copy(
        points_hbm.at[pl.ds(base * 2, T * 2)], bufs[t % 2], sems[t % 2])

  handles = [None, None]
  handles[0] = _start(0)

  for t in range(tiles_per_worker):
    b = t % 2
    if t + 1 < tiles_per_worker:
      handles[(t + 1) % 2] = _start(t + 1)
    handles[b].wait()

    tile_start = (wid + t * nw) * T
    base = jnp.minimum(tile_start, last_base)
    off = tile_start - base
    buf = bufs[b]

    def _group(g, c, buf=buf, off=off):
      local = g * L + iota
      ix = local * 2
      xv = plsc.load_gather(buf, [ix])
      yv = plsc.load_gather(buf, [ix + 1])
      xv = jnp.clip(xv, 0.0, CLIP_MAX)
      yv = jnp.clip(yv, 0.0, CLIP_MAX)
      xi = (xv + 0.5).astype(jnp.int32)
      yi = (yv + 0.5).astype(jnp.int32)
      flat = xi * GRID_N + yi
      valid = local >= off
      counts, last_occ = plsc.scan_count(flat, mask=valid)
      plsc.addupdate_scatter(
          hist, [flat], counts.astype(jnp.float32), mask=last_occ)
      return c
    lax.fori_loop(0, groups, _group, 0)

  pltpu.sync_copy(hist, out_hbm.at[wid])


def _merge_body(partials_ref, grid_ref, out_ref):
  s = jnp.sum(partials_ref[...], axis=0)
  prob = s / NORM
  out_ref[...] = jnp.where(grid_ref[...] != 0, 0.0, prob)


@jax.jit
def kernel(all_points, grid):
  pts_flat = all_points.reshape(-1)
  grid_flat = grid.reshape(-1)
  sc_hist = pl.kernel(
      _sc_hist_body,
      out_type=jax.ShapeDtypeStruct((NC * NS, NBINS), jnp.float32),
      mesh=plsc.VectorSubcoreMesh(
          core_axis_name="c", subcore_axis_name="s",
          num_cores=NC, num_subcores=NS),
      compiler_params=pltpu.CompilerParams(needs_layout_passes=False),
      scratch_types=[
          pltpu.VMEM((T * 2,), jnp.float32),
          pltpu.VMEM((T * 2,), jnp.float32),
          pltpu.VMEM((NBINS,), jnp.float32),
          pltpu.SemaphoreType.DMA,
          pltpu.SemaphoreType.DMA,
      ],
  )
  partials = sc_hist(pts_flat)
  merge = pl.pallas_call(
      _merge_body,
      out_shape=jax.ShapeDtypeStruct((NBINS,), jnp.float32),
  )
  return merge(partials, grid_flat).reshape(GRID_N, GRID_N)


# speedup vs baseline: 75.7044x; 19.7068x over previous
import jax
import jax.numpy as jnp
import numpy as np
from jax import lax
from jax.experimental import pallas as pl
from jax.experimental.pallas import tpu as pltpu
from jax.experimental.pallas import tpu_sc as plsc

GRID_N = 200
NBINS = GRID_N * GRID_N
NORM = float(25000 * 80)
CLIP_MAX = np.float32(GRID_N - 1 - 1e-6)

NC = 2
NS = 16
L = 16

T = 8192
UNROLL = 4


def _sc_hist_body(xs_hbm, ys_hbm, out_hbm,
                  bufxa, bufya, bufxb, bufyb, hist,
                  semxa, semya, semxb, semyb):
  nw = NC * NS
  n = xs_hbm.shape[0]
  num_tiles = -(-n // T)
  tiles_per_worker = -(-num_tiles // nw)
  last_base = n - T

  wid = lax.axis_index("c") * NS + lax.axis_index("s")

  iota = lax.iota(jnp.int32, L)

  def _zero(i, c):
    hist[pl.ds(i * L, L)] = jnp.zeros((L,), jnp.float32)
    return c
  lax.fori_loop(0, NBINS // L, _zero, 0)

  bufs = ((bufxa, bufya), (bufxb, bufyb))
  sems = ((semxa, semya), (semxb, semyb))

  def _start(t):
    base = jnp.minimum((wid + t * nw) * T, last_base)
    s = t % 2
    hx = pltpu.async_copy(xs_hbm.at[pl.ds(base, T)], bufs[s][0], sems[s][0])
    hy = pltpu.async_copy(ys_hbm.at[pl.ds(base, T)], bufs[s][1], sems[s][1])
    return hx, hy

  handles = [None, None]
  handles[0] = _start(0)

  for t in range(tiles_per_worker):
    b = t % 2
    if t + 1 < tiles_per_worker:
      handles[(t + 1) % 2] = _start(t + 1)
    handles[b][0].wait()
    handles[b][1].wait()

    tile_start = (wid + t * nw) * T
    base = jnp.minimum(tile_start, last_base)
    off = tile_start - base
    bufx, bufy = bufs[b]

    def _group(g, bufx=bufx, bufy=bufy, off=off):
      lo = g * L
      xv = bufx[pl.ds(lo, L)]
      yv = bufy[pl.ds(lo, L)]
      xv = jnp.clip(xv, 0.0, CLIP_MAX)
      yv = jnp.clip(yv, 0.0, CLIP_MAX)
      xi = (xv + 0.5).astype(jnp.int32)
      yi = (yv + 0.5).astype(jnp.int32)
      flat = xi * GRID_N + yi
      valid = (lo + iota) >= off
      counts, last_occ = plsc.scan_count(flat, mask=valid)
      plsc.addupdate_scatter(
          hist, [flat], counts.astype(jnp.float32), mask=last_occ)

    plsc.parallel_loop(0, T // L, 1, unroll=UNROLL)(_group)

  pltpu.sync_copy(hist, out_hbm.at[wid])


def _merge_body(partials_ref, grid_ref, out_ref):
  s = jnp.sum(partials_ref[...], axis=0)
  prob = s / NORM
  out_ref[...] = jnp.where(grid_ref[...] != 0, 0.0, prob)


@jax.jit
def kernel(all_points, grid):
  xs = all_points[:, 0]
  ys = all_points[:, 1]
  grid_flat = grid.reshape(-1)
  sc_hist = pl.kernel(
      _sc_hist_body,
      out_type=jax.ShapeDtypeStruct((NC * NS, NBINS), jnp.float32),
      mesh=plsc.VectorSubcoreMesh(
          core_axis_name="c", subcore_axis_name="s",
          num_cores=NC, num_subcores=NS),
      compiler_params=pltpu.CompilerParams(needs_layout_passes=False),
      scratch_types=[
          pltpu.VMEM((T,), jnp.float32),
          pltpu.VMEM((T,), jnp.float32),
          pltpu.VMEM((T,), jnp.float32),
          pltpu.VMEM((T,), jnp.float32),
          pltpu.VMEM((NBINS,), jnp.float32),
          pltpu.SemaphoreType.DMA,
          pltpu.SemaphoreType.DMA,
          pltpu.SemaphoreType.DMA,
          pltpu.SemaphoreType.DMA,
      ],
  )
  partials = sc_hist(xs, ys)
  merge = pl.pallas_call(
      _merge_body,
      out_shape=jax.ShapeDtypeStruct((NBINS,), jnp.float32),
  )
  return merge(partials, grid_flat).reshape(GRID_N, GRID_N)


# speedup vs baseline: 79.5521x; 1.0508x over previous
import jax
import jax.numpy as jnp
import numpy as np
from jax import lax
from jax.experimental import pallas as pl
from jax.experimental.pallas import tpu as pltpu
from jax.experimental.pallas import tpu_sc as plsc

GRID_N = 200
NBINS = GRID_N * GRID_N
NORM = float(25000 * 80)
CLIP_MAX = np.float32(GRID_N - 1 - 1e-6)

NC = 2
NS = 16
L = 16

T = 8192
UNROLL = 4


def _sc_hist_body(bins_hbm, out_hbm, bufa, bufb, hist, sema, semb):
  nw = NC * NS
  n = bins_hbm.shape[0]
  num_tiles = -(-n // T)
  tiles_per_worker = -(-num_tiles // nw)
  last_base = n - T

  wid = lax.axis_index("c") * NS + lax.axis_index("s")

  iota = lax.iota(jnp.int32, L)

  def _zero(i, c):
    hist[pl.ds(i * L, L)] = jnp.zeros((L,), jnp.float32)
    return c
  lax.fori_loop(0, NBINS // L, _zero, 0)

  bufs = (bufa, bufb)
  sems = (sema, semb)

  def _start(t):
    base = jnp.minimum((wid + t * nw) * T, last_base)
    s = t % 2
    return pltpu.async_copy(bins_hbm.at[pl.ds(base, T)], bufs[s], sems[s])

  handles = [None, None]
  handles[0] = _start(0)

  for t in range(tiles_per_worker):
    b = t % 2
    if t + 1 < tiles_per_worker:
      handles[(t + 1) % 2] = _start(t + 1)
    handles[b].wait()

    tile_start = (wid + t * nw) * T
    base = jnp.minimum(tile_start, last_base)
    off = tile_start - base
    buf = bufs[b]

    def _group(g, buf=buf, off=off):
      lo = g * L
      flat = buf[pl.ds(lo, L)]
      valid = (lo + iota) >= off
      counts, last_occ = plsc.scan_count(flat, mask=valid)
      plsc.addupdate_scatter(
          hist, [flat], counts.astype(jnp.float32), mask=last_occ)

    plsc.parallel_loop(0, T // L, 1, unroll=UNROLL)(_group)

  pltpu.sync_copy(hist, out_hbm.at[wid])


def _merge_body(partials_ref, grid_ref, out_ref):
  s = jnp.sum(partials_ref[...], axis=0)
  prob = s / NORM
  out_ref[...] = jnp.where(grid_ref[...] != 0, 0.0, prob)


@jax.jit
def kernel(all_points, grid):
  pts = jnp.clip(all_points, 0.0, CLIP_MAX)
  idx = (pts + 0.5).astype(jnp.int32)
  w = jnp.array([GRID_N, 1], jnp.int32)
  flat_bins = jnp.sum(idx * w[None, :], axis=1)

  grid_flat = grid.reshape(-1)
  sc_hist = pl.kernel(
      _sc_hist_body,
      out_type=jax.ShapeDtypeStruct((NC * NS, NBINS), jnp.float32),
      mesh=plsc.VectorSubcoreMesh(
          core_axis_name="c", subcore_axis_name="s",
          num_cores=NC, num_subcores=NS),
      compiler_params=pltpu.CompilerParams(needs_layout_passes=False),
      scratch_types=[
          pltpu.VMEM((T,), jnp.int32),
          pltpu.VMEM((T,), jnp.int32),
          pltpu.VMEM((NBINS,), jnp.float32),
          pltpu.SemaphoreType.DMA,
          pltpu.SemaphoreType.DMA,
      ],
  )
  partials = sc_hist(flat_bins)
  merge = pl.pallas_call(
      _merge_body,
      out_shape=jax.ShapeDtypeStruct((NBINS,), jnp.float32),
  )
  return merge(partials, grid_flat).reshape(GRID_N, GRID_N)


# speedup vs baseline: 83.8041x; 1.0534x over previous
import jax
import jax.numpy as jnp
import numpy as np
from jax import lax
from jax.experimental import pallas as pl
from jax.experimental.pallas import tpu as pltpu
from jax.experimental.pallas import tpu_sc as plsc

GRID_N = 200
NBINS = GRID_N * GRID_N
NORM = float(25000 * 80)
CLIP_MAX = np.float32(GRID_N - 1 - 1e-6)

NC = 2
NS = 16
L = 16
BLK = 128

T = 8192
UNROLL = 4
TAIL_PAD = 48


def _make_sc_body(n_tail_valid):
  def _sc_hist_body(zflat_hbm, tail_hbm, out_hbm,
                    bufa, bufb, tailbuf, hist, sema, semb, semt):
    nw = NC * NS
    nh = zflat_hbm.shape[0] // 2
    num_tiles = -(-nh // T)
    tiles_per_worker = -(-num_tiles // nw)
    last_base = nh - T

    wid = lax.axis_index("c") * NS + lax.axis_index("s")

    iota = lax.iota(jnp.int32, L)

    def _zero(i, c):
      hist[pl.ds(i * L, L)] = jnp.zeros((L,), jnp.float32)
      return c
    lax.fori_loop(0, NBINS // L, _zero, 0)

    bufs = (bufa, bufb)
    sems = (sema, semb)

    tail_handle = pltpu.async_copy(tail_hbm, tailbuf, semt)

    def _start(t):
      base = jnp.minimum((wid + t * nw) * T, last_base)
      s = t % 2
      return pltpu.async_copy(
          zflat_hbm.at[pl.ds(base * 2, T * 2)], bufs[s], sems[s])

    handles = [None, None]
    handles[0] = _start(0)

    for t in range(tiles_per_worker):
      b = t % 2
      if t + 1 < tiles_per_worker:
        handles[(t + 1) % 2] = _start(t + 1)
      handles[b].wait()

      tile_start = (wid + t * nw) * T
      base = jnp.minimum(tile_start, last_base)
      off = tile_start - base
      buf = bufs[b]

      def _group(g, buf=buf, off=off):
        lo = (g >> 3) * (2 * BLK) + (g & 7) * L
        xv = buf[pl.ds(lo, L)]
        yv = buf[pl.ds(lo + BLK, L)]
        flat = xv + yv
        valid = (g * L + iota) >= off
        counts, last_occ = plsc.scan_count(flat, mask=valid)
        plsc.addupdate_scatter(
            hist, [flat], counts.astype(jnp.float32), mask=last_occ)

      plsc.parallel_loop(0, T // L, 1, unroll=UNROLL)(_group)

    tail_handle.wait()
    if n_tail_valid:
      @pl.when(wid == 0)
      def _tail():
        for g in range(TAIL_PAD // L):
          flat = tailbuf[pl.ds(g * L, L)]
          valid = (g * L + iota) < n_tail_valid
          counts, last_occ = plsc.scan_count(flat, mask=valid)
          plsc.addupdate_scatter(
              hist, [flat], counts.astype(jnp.float32), mask=last_occ)

    pltpu.sync_copy(hist, out_hbm.at[wid])

  return _sc_hist_body


def _merge_body(partials_ref, grid_ref, out_ref):
  s = jnp.sum(partials_ref[...], axis=0)
  prob = s / NORM
  out_ref[...] = jnp.where(grid_ref[...] != 0, 0.0, prob)


@jax.jit
def kernel(all_points, grid):
  n = all_points.shape[0]
  nb = n // BLK
  nh = nb * BLK
  n_tail = n - nh

  q = jnp.clip(all_points, 0.0, CLIP_MAX)
  b = (q + 0.5).astype(jnp.int32) * jnp.array([GRID_N, 1], jnp.int32)[None, :]

  zflat = b[:nh].reshape(nb, BLK, 2).transpose(0, 2, 1).reshape(-1)
  tail = b[nh:, 0] + b[nh:, 1]
  tail = jnp.pad(tail, (0, TAIL_PAD - n_tail), constant_values=0)

  grid_flat = grid.reshape(-1)
  sc_hist = pl.kernel(
      _make_sc_body(n_tail),
      out_type=jax.ShapeDtypeStruct((NC * NS, NBINS), jnp.float32),
      mesh=plsc.VectorSubcoreMesh(
          core_axis_name="c", subcore_axis_name="s",
          num_cores=NC, num_subcores=NS),
      compiler_params=pltpu.CompilerParams(needs_layout_passes=False),
      scratch_types=[
          pltpu.VMEM((T * 2,), jnp.int32),
          pltpu.VMEM((T * 2,), jnp.int32),
          pltpu.VMEM((TAIL_PAD,), jnp.int32),
          pltpu.VMEM((NBINS,), jnp.float32),
          pltpu.SemaphoreType.DMA,
          pltpu.SemaphoreType.DMA,
          pltpu.SemaphoreType.DMA,
      ],
  )
  partials = sc_hist(zflat, tail)
  merge = pl.pallas_call(
      _merge_body,
      out_shape=jax.ShapeDtypeStruct((NBINS,), jnp.float32),
  )
  return merge(partials, grid_flat).reshape(GRID_N, GRID_N)


# speedup vs baseline: 118.3509x; 1.4122x over previous
import jax
import jax.numpy as jnp
import numpy as np
from jax import lax
from jax.experimental import pallas as pl
from jax.experimental.pallas import tpu as pltpu
from jax.experimental.pallas import tpu_sc as plsc

GRID_N = 200
NBINS = GRID_N * GRID_N
NORM = float(25000 * 80)
CLIP_MAX = np.float32(GRID_N - 1 - 1e-6)

NC = 2
NS = 16
L = 16
BLK = 128

T = 8192
UNROLL = 4


def _make_sc_body(n_tail_valid, tail_pad):
  def _sc_hist_body(zp_hbm, tail_hbm, out_hbm,
                    bufxa, bufya, bufxb, bufyb, tailbuf, hist,
                    semxa, semya, semxb, semyb, semt):
    nw = NC * NS
    nh = zp_hbm.shape[0] // 2
    num_tiles = -(-nh // T)
    tiles_per_worker = -(-num_tiles // nw)
    last_base = nh - T

    wid = lax.axis_index("c") * NS + lax.axis_index("s")

    iota = lax.iota(jnp.int32, L)

    def _zero(i, c):
      hist[pl.ds(i * L, L)] = jnp.zeros((L,), jnp.float32)
      return c
    lax.fori_loop(0, NBINS // L, _zero, 0)

    bufs = ((bufxa, bufya), (bufxb, bufyb))
    sems = ((semxa, semya), (semxb, semyb))

    tail_handle = pltpu.async_copy(tail_hbm, tailbuf, semt)

    def _start(t):
      base = jnp.minimum((wid + t * nw) * T, last_base)
      s = t % 2
      hx = pltpu.async_copy(zp_hbm.at[pl.ds(base, T)], bufs[s][0], sems[s][0])
      hy = pltpu.async_copy(
          zp_hbm.at[pl.ds(nh + base, T)], bufs[s][1], sems[s][1])
      return hx, hy

    handles = [None, None]
    handles[0] = _start(0)

    for t in range(tiles_per_worker):
      b = t % 2
      if t + 1 < tiles_per_worker:
        handles[(t + 1) % 2] = _start(t + 1)
      handles[b][0].wait()
      handles[b][1].wait()

      tile_start = (wid + t * nw) * T
      base = jnp.minimum(tile_start, last_base)
      off = tile_start - base
      bufx, bufy = bufs[b]

      def _group(g, bufx=bufx, bufy=bufy, off=off):
        lo = g * L
        flat = bufx[pl.ds(lo, L)] + bufy[pl.ds(lo, L)]
        valid = (lo + iota) >= off
        counts, last_occ = plsc.scan_count(flat, mask=valid)
        plsc.addupdate_scatter(
            hist, [flat], counts.astype(jnp.float32), mask=last_occ)

      plsc.parallel_loop(0, T // L, 1, unroll=UNROLL)(_group)

    tail_handle.wait()
    if n_tail_valid:
      @pl.when(wid == 0)
      def _tail():
        def _tgroup(g):
          flat = tailbuf[pl.ds(g * L, L)]
          valid = (g * L + iota) < n_tail_valid
          counts, last_occ = plsc.scan_count(flat, mask=valid)
          plsc.addupdate_scatter(
              hist, [flat], counts.astype(jnp.float32), mask=last_occ)
        plsc.parallel_loop(0, tail_pad // L, 1, unroll=1)(_tgroup)

    pltpu.sync_copy(hist, out_hbm.at[wid])

  return _sc_hist_body


def _merge_body(partials_ref, grid_ref, out_ref):
  s = jnp.sum(partials_ref[...], axis=0)
  prob = s / NORM
  out_ref[...] = jnp.where(grid_ref[...] != 0, 0.0, prob)


@jax.jit
def kernel(all_points, grid):
  n = all_points.shape[0]
  nb = (n // (8 * BLK)) * 8
  nh = nb * BLK
  n_tail = n - nh
  tail_pad = -(-n_tail // L) * L if n_tail else 0

  q = jnp.clip(all_points, 0.0, CLIP_MAX)
  b = (q + 0.5).astype(jnp.int32) * jnp.array([GRID_N, 1], jnp.int32)[None, :]

  zp = b[:nh].reshape(nb, BLK, 2).transpose(2, 0, 1).reshape(-1)
  args = [zp]
  scratch = [
      pltpu.VMEM((T,), jnp.int32),
      pltpu.VMEM((T,), jnp.int32),
      pltpu.VMEM((T,), jnp.int32),
      pltpu.VMEM((T,), jnp.int32),
      pltpu.VMEM((max(tail_pad, L),), jnp.int32),
      pltpu.VMEM((NBINS,), jnp.float32),
      pltpu.SemaphoreType.DMA,
      pltpu.SemaphoreType.DMA,
      pltpu.SemaphoreType.DMA,
      pltpu.SemaphoreType.DMA,
      pltpu.SemaphoreType.DMA,
  ]
  tail = b[nh:, 0] + b[nh:, 1]
  tail = jnp.pad(tail, (0, max(tail_pad, L) - n_tail), constant_values=0)
  args.append(tail)

  grid_flat = grid.reshape(-1)
  sc_hist = pl.kernel(
      _make_sc_body(n_tail, max(tail_pad, L)),
      out_type=jax.ShapeDtypeStruct((NC * NS, NBINS), jnp.float32),
      mesh=plsc.VectorSubcoreMesh(
          core_axis_name="c", subcore_axis_name="s",
          num_cores=NC, num_subcores=NS),
      compiler_params=pltpu.CompilerParams(needs_layout_passes=False),
      scratch_types=scratch,
  )
  partials = sc_hist(*args)
  merge = pl.pallas_call(
      _merge_body,
      out_shape=jax.ShapeDtypeStruct((NBINS,), jnp.float32),
  )
  return merge(partials, grid_flat).reshape(GRID_N, GRID_N)


# speedup vs baseline: 119.1351x; 1.0066x over previous
import jax
import jax.numpy as jnp
import numpy as np
from jax import lax
from jax.experimental import pallas as pl
from jax.experimental.pallas import tpu as pltpu
from jax.experimental.pallas import tpu_sc as plsc

GRID_N = 200
NBINS = GRID_N * GRID_N
NORM = float(25000 * 80)
CLIP_MAX = np.float32(GRID_N - 1 - 1e-6)

NC = 2
NS = 16
L = 16
BLK = 128

T = 8192
UNROLL = 4


def _make_sc_body(n_tail_valid, tail_pad):
  def _sc_hist_body(zp_hbm, tail_hbm, out_hbm,
                    bufxa, bufya, bufxb, bufyb, tailbuf, hist,
                    semxa, semya, semxb, semyb, semt):
    nw = NC * NS
    nh = zp_hbm.shape[0] // 2
    num_tiles = -(-nh // T)
    tiles_per_worker = -(-num_tiles // nw)
    last_base = nh - T

    wid = lax.axis_index("c") * NS + lax.axis_index("s")

    iota = lax.iota(jnp.int32, L)
    ones_f = jnp.ones((L,), jnp.float32)

    def _zero(i, c):
      hist[pl.ds(i * L, L)] = jnp.zeros((L,), jnp.float32)
      return c
    lax.fori_loop(0, NBINS // L, _zero, 0)

    bufs = ((bufxa, bufya), (bufxb, bufyb))
    sems = ((semxa, semya), (semxb, semyb))

    tail_handle = pltpu.async_copy(tail_hbm, tailbuf, semt)

    def _start(t):
      base = jnp.minimum((wid + t * nw) * T, last_base)
      s = t % 2
      hx = pltpu.async_copy(zp_hbm.at[pl.ds(base, T)], bufs[s][0], sems[s][0])
      hy = pltpu.async_copy(
          zp_hbm.at[pl.ds(nh + base, T)], bufs[s][1], sems[s][1])
      return hx, hy

    handles = [None, None]
    handles[0] = _start(0)

    for t in range(tiles_per_worker):
      b = t % 2
      if t + 1 < tiles_per_worker:
        handles[(t + 1) % 2] = _start(t + 1)
      handles[b][0].wait()
      handles[b][1].wait()

      tile_start = (wid + t * nw) * T
      base = jnp.minimum(tile_start, last_base)
      off = tile_start - base
      bufx, bufy = bufs[b]

      def _group(g, bufx=bufx, bufy=bufy, off=off):
        lo = g * L
        flat = bufx[pl.ds(lo, L)] + bufy[pl.ds(lo, L)]
        valid = (lo + iota) >= off
        plsc.addupdate_scatter(hist, [flat], ones_f, mask=valid)

      plsc.parallel_loop(0, T // L, 1, unroll=UNROLL)(_group)

    tail_handle.wait()
    if n_tail_valid:
      @pl.when(wid == 0)
      def _tail():
        def _tgroup(g):
          flat = tailbuf[pl.ds(g * L, L)]
          valid = (g * L + iota) < n_tail_valid
          plsc.addupdate_scatter(hist, [flat], ones_f, mask=valid)
        plsc.parallel_loop(0, tail_pad // L, 1, unroll=1)(_tgroup)

    pltpu.sync_copy(hist, out_hbm.at[wid])

  return _sc_hist_body


def _merge_body(partials_ref, grid_ref, out_ref):
  s = jnp.sum(partials_ref[...], axis=0)
  prob = s / NORM
  out_ref[...] = jnp.where(grid_ref[...] != 0, 0.0, prob)


@jax.jit
def kernel(all_points, grid):
  n = all_points.shape[0]
  nb = (n // (8 * BLK)) * 8
  nh = nb * BLK
  n_tail = n - nh
  tail_pad = -(-n_tail // L) * L if n_tail else 0

  q = jnp.clip(all_points, 0.0, CLIP_MAX)
  b = (q + 0.5).astype(jnp.int32) * jnp.array([GRID_N, 1], jnp.int32)[None, :]

  zp = b[:nh].reshape(nb, BLK, 2).transpose(2, 0, 1).reshape(-1)
  args = [zp]
  scratch = [
      pltpu.VMEM((T,), jnp.int32),
      pltpu.VMEM((T,), jnp.int32),
      pltpu.VMEM((T,), jnp.int32),
      pltpu.VMEM((T,), jnp.int32),
      pltpu.VMEM((max(tail_pad, L),), jnp.int32),
      pltpu.VMEM((NBINS,), jnp.float32),
      pltpu.SemaphoreType.DMA,
      pltpu.SemaphoreType.DMA,
      pltpu.SemaphoreType.DMA,
      pltpu.SemaphoreType.DMA,
      pltpu.SemaphoreType.DMA,
  ]
  tail = b[nh:, 0] + b[nh:, 1]
  tail = jnp.pad(tail, (0, max(tail_pad, L) - n_tail), constant_values=0)
  args.append(tail)

  grid_flat = grid.reshape(-1)
  sc_hist = pl.kernel(
      _make_sc_body(n_tail, max(tail_pad, L)),
      out_type=jax.ShapeDtypeStruct((NC * NS, NBINS), jnp.float32),
      mesh=plsc.VectorSubcoreMesh(
          core_axis_name="c", subcore_axis_name="s",
          num_cores=NC, num_subcores=NS),
      compiler_params=pltpu.CompilerParams(needs_layout_passes=False),
      scratch_types=scratch,
  )
  partials = sc_hist(*args)
  merge = pl.pallas_call(
      _merge_body,
      out_shape=jax.ShapeDtypeStruct((NBINS,), jnp.float32),
  )
  return merge(partials, grid_flat).reshape(GRID_N, GRID_N)


# speedup vs baseline: 136.3312x; 1.1443x over previous
import jax
import jax.numpy as jnp
import numpy as np
from jax import lax
from jax.experimental import pallas as pl
from jax.experimental.pallas import tpu as pltpu
from jax.experimental.pallas import tpu_sc as plsc

GRID_N = 200
NBINS = GRID_N * GRID_N
NORM = float(25000 * 80)
CLIP_MAX = np.float32(GRID_N - 1 - 1e-6)

NC = 2
NS = 16
L = 16
BLK = 128

T = 8192
UNROLL = 8


def _make_sc_body(n_tail_valid, tail_pad):
  def _sc_hist_body(zp_hbm, tail_hbm, out_hbm,
                    bufxa, bufya, bufxb, bufyb, tailbuf, hist,
                    semxa, semya, semxb, semyb, semt):
    nw = NC * NS
    nh = zp_hbm.shape[0] // 2
    num_tiles = -(-nh // T)
    tiles_per_worker = -(-num_tiles // nw)
    last_base = nh - T

    wid = lax.axis_index("c") * NS + lax.axis_index("s")

    iota = lax.iota(jnp.int32, L)
    ones_f = jnp.ones((L,), jnp.float32)

    bufs = ((bufxa, bufya), (bufxb, bufyb))
    sems = ((semxa, semya), (semxb, semyb))

    tail_handle = pltpu.async_copy(tail_hbm, tailbuf, semt)

    def _start(t):
      base = jnp.minimum((wid + t * nw) * T, last_base)
      s = t % 2
      hx = pltpu.async_copy(zp_hbm.at[pl.ds(base, T)], bufs[s][0], sems[s][0])
      hy = pltpu.async_copy(
          zp_hbm.at[pl.ds(nh + base, T)], bufs[s][1], sems[s][1])
      return hx, hy

    handles = [None, None]
    handles[0] = _start(0)

    zeros_f = jnp.zeros((L,), jnp.float32)

    def _zero(i):
      hist[pl.ds(i * L, L)] = zeros_f
    plsc.parallel_loop(0, NBINS // L, 1, unroll=8)(_zero)

    for t in range(tiles_per_worker):
      b = t % 2
      if t + 1 < tiles_per_worker:
        handles[(t + 1) % 2] = _start(t + 1)
      handles[b][0].wait()
      handles[b][1].wait()

      tile_start = (wid + t * nw) * T
      base = jnp.minimum(tile_start, last_base)
      off = tile_start - base
      bufx, bufy = bufs[b]

      if t + 1 < tiles_per_worker:
        def _group(g, bufx=bufx, bufy=bufy):
          lo = g * L
          flat = bufx[pl.ds(lo, L)] + bufy[pl.ds(lo, L)]
          plsc.addupdate_scatter(hist, [flat], ones_f)
      else:
        def _group(g, bufx=bufx, bufy=bufy, off=off):
          lo = g * L
          flat = bufx[pl.ds(lo, L)] + bufy[pl.ds(lo, L)]
          valid = (lo + iota) >= off
          plsc.addupdate_scatter(hist, [flat], ones_f, mask=valid)

      plsc.parallel_loop(0, T // L, 1, unroll=UNROLL)(_group)

    tail_handle.wait()
    if n_tail_valid:
      @pl.when(wid == 0)
      def _tail():
        def _tgroup(g):
          flat = tailbuf[pl.ds(g * L, L)]
          valid = (g * L + iota) < n_tail_valid
          plsc.addupdate_scatter(hist, [flat], ones_f, mask=valid)
        plsc.parallel_loop(0, tail_pad // L, 1, unroll=1)(_tgroup)

    pltpu.sync_copy(hist, out_hbm.at[wid])

  return _sc_hist_body


def _merge_body(partials_ref, grid_ref, out_ref):
  s = jnp.sum(partials_ref[...], axis=0)
  prob = s / NORM
  out_ref[...] = jnp.where(grid_ref[...] != 0, 0.0, prob)


@jax.jit
def kernel(all_points, grid):
  n = all_points.shape[0]
  nb = (n // (8 * BLK)) * 8
  nh = nb * BLK
  n_tail = n - nh
  tail_pad = -(-n_tail // L) * L if n_tail else 0

  q = jnp.clip(all_points, 0.0, CLIP_MAX)
  b = (q + 0.5).astype(jnp.int32) * jnp.array([GRID_N, 1], jnp.int32)[None, :]

  zp = b[:nh].reshape(nb, BLK, 2).transpose(2, 0, 1).reshape(-1)
  args = [zp]
  scratch = [
      pltpu.VMEM((T,), jnp.int32),
      pltpu.VMEM((T,), jnp.int32),
      pltpu.VMEM((T,), jnp.int32),
      pltpu.VMEM((T,), jnp.int32),
      pltpu.VMEM((max(tail_pad, L),), jnp.int32),
      pltpu.VMEM((NBINS,), jnp.float32),
      pltpu.SemaphoreType.DMA,
      pltpu.SemaphoreType.DMA,
      pltpu.SemaphoreType.DMA,
      pltpu.SemaphoreType.DMA,
      pltpu.SemaphoreType.DMA,
  ]
  tail = b[nh:, 0] + b[nh:, 1]
  tail = jnp.pad(tail, (0, max(tail_pad, L) - n_tail), constant_values=0)
  args.append(tail)

  grid_flat = grid.reshape(-1)
  sc_hist = pl.kernel(
      _make_sc_body(n_tail, max(tail_pad, L)),
      out_type=jax.ShapeDtypeStruct((NC * NS, NBINS), jnp.float32),
      mesh=plsc.VectorSubcoreMesh(
          core_axis_name="c", subcore_axis_name="s",
          num_cores=NC, num_subcores=NS),
      compiler_params=pltpu.CompilerParams(needs_layout_passes=False),
      scratch_types=scratch,
  )
  partials = sc_hist(*args)
  merge = pl.pallas_call(
      _merge_body,
      out_shape=jax.ShapeDtypeStruct((NBINS,), jnp.float32),
  )
  return merge(partials, grid_flat).reshape(GRID_N, GRID_N)


# speedup vs baseline: 206.0949x; 1.5117x over previous
import jax
import jax.numpy as jnp
import numpy as np
from jax import lax
from jax.experimental import pallas as pl
from jax.experimental.pallas import tpu as pltpu
from jax.experimental.pallas import tpu_sc as plsc

GRID_N = 200
NBINS = GRID_N * GRID_N
NORM = float(25000 * 80)
CLIP_MAX = np.float32(GRID_N - 1 - 1e-6)

NC = 2
NS = 16
L = 16
BLK = 128

TB = 64
T = TB * BLK
UNROLL = 8


def _bin16(xv, yv):
  xv = jnp.clip(xv, 0.0, CLIP_MAX)
  yv = jnp.clip(yv, 0.0, CLIP_MAX)
  xi = (xv + 0.5).astype(jnp.int32)
  yi = (yv + 0.5).astype(jnp.int32)
  return xi * GRID_N + yi


def _make_sc_body(n_tail_valid, tail_pad):
  def _sc_hist_body(z_hbm, tail_hbm, out_hbm,
                    bufa, bufb, tailbuf, hist, sema, semb, semt):
    nw = NC * NS
    nb = z_hbm.shape[0]
    num_tiles = -(-nb // TB)
    tiles_per_worker = -(-num_tiles // nw)
    last_base = nb - TB

    wid = lax.axis_index("c") * NS + lax.axis_index("s")

    iota = lax.iota(jnp.int32, L)
    ones_f = jnp.ones((L,), jnp.float32)

    bufs = (bufa, bufb)
    sems = (sema, semb)

    tail_handle = pltpu.async_copy(tail_hbm, tailbuf, semt)

    def _start(t):
      base = jnp.minimum((wid + t * nw) * TB, last_base)
      s = t % 2
      return pltpu.async_copy(z_hbm.at[pl.ds(base, TB)], bufs[s], sems[s])

    handles = [None, None]
    handles[0] = _start(0)

    zeros_f = jnp.zeros((L,), jnp.float32)

    def _zero(i):
      hist[pl.ds(i * L, L)] = zeros_f
    plsc.parallel_loop(0, NBINS // L, 1, unroll=8)(_zero)

    for t in range(tiles_per_worker):
      b = t % 2
      if t + 1 < tiles_per_worker:
        handles[(t + 1) % 2] = _start(t + 1)
      handles[b].wait()

      tile_start = (wid + t * nw) * T
      base_pts = jnp.minimum(tile_start, last_base * BLK)
      off = tile_start - base_pts
      buf = bufs[b]

      if t + 1 < tiles_per_worker:
        def _group(g, buf=buf):
          blk = g >> 3
          j = (g & 7) * L
          flat = _bin16(buf[blk, 0, pl.ds(j, L)], buf[blk, 1, pl.ds(j, L)])
          plsc.addupdate_scatter(hist, [flat], ones_f)
      else:
        def _group(g, buf=buf, off=off):
          blk = g >> 3
          j = (g & 7) * L
          flat = _bin16(buf[blk, 0, pl.ds(j, L)], buf[blk, 1, pl.ds(j, L)])
          valid = (g * L + iota) >= off
          plsc.addupdate_scatter(hist, [flat], ones_f, mask=valid)

      plsc.parallel_loop(0, T // L, 1, unroll=UNROLL)(_group)

    tail_handle.wait()
    if n_tail_valid:
      @pl.when(wid == 0)
      def _tail():
        def _tgroup(g):
          flat = _bin16(tailbuf[0, pl.ds(g * L, L)],
                        tailbuf[1, pl.ds(g * L, L)])
          valid = (g * L + iota) < n_tail_valid
          plsc.addupdate_scatter(hist, [flat], ones_f, mask=valid)
        plsc.parallel_loop(0, tail_pad // L, 1, unroll=1)(_tgroup)

    pltpu.sync_copy(hist, out_hbm.at[wid])

  return _sc_hist_body


def _merge_body(partials_ref, grid_ref, out_ref):
  s = jnp.sum(partials_ref[...], axis=0)
  prob = s / NORM
  out_ref[...] = jnp.where(grid_ref[...] != 0, 0.0, prob)


@jax.jit
def kernel(all_points, grid):
  n = all_points.shape[0]
  nb = n // BLK
  nh = nb * BLK
  n_tail = n - nh
  tail_pad = -(-n_tail // L) * L if n_tail else L

  z3 = all_points[:nh].reshape(nb, BLK, 2).transpose(0, 2, 1)
  tail = jnp.pad(all_points[nh:].T, ((0, 0), (0, tail_pad - n_tail)))

  grid_flat = grid.reshape(-1)
  sc_hist = pl.kernel(
      _make_sc_body(n_tail, tail_pad),
      out_type=jax.ShapeDtypeStruct((NC * NS, NBINS), jnp.float32),
      mesh=plsc.VectorSubcoreMesh(
          core_axis_name="c", subcore_axis_name="s",
          num_cores=NC, num_subcores=NS),
      compiler_params=pltpu.CompilerParams(needs_layout_passes=False),
      scratch_types=[
          pltpu.VMEM((TB, 2, BLK), jnp.float32),
          pltpu.VMEM((TB, 2, BLK), jnp.float32),
          pltpu.VMEM((2, tail_pad), jnp.float32),
          pltpu.VMEM((NBINS,), jnp.float32),
          pltpu.SemaphoreType.DMA,
          pltpu.SemaphoreType.DMA,
          pltpu.SemaphoreType.DMA,
      ],
  )
  partials = sc_hist(z3, tail)
  merge = pl.pallas_call(
      _merge_body,
      out_shape=jax.ShapeDtypeStruct((NBINS,), jnp.float32),
  )
  return merge(partials, grid_flat).reshape(GRID_N, GRID_N)
